# Initial kernel scaffold; baseline (speedup 1.0000x reference)
import jax
import jax.numpy as jnp
from jax.experimental import pallas as pl


def kernel(features, edge_index, W_self1, W_neigh1, b1, W_self2, W_neigh2, b2, gate_W, gate_b):
    raise NotImplementedError("write your pallas kernel here")


# speedup vs baseline: 3.7574x; 3.7574x over previous
import functools

import jax
import jax.numpy as jnp
from jax import lax
from jax.experimental import pallas as pl
from jax.experimental.pallas import tpu as pltpu
from jax.experimental.pallas import tpu_sc as plsc

N = 10000
E = 320000
D_IN = 128
D_HID = 128
D_OUT = 64

NC = 2
NS = 16
NW = NC * NS

NPAD = 10240
ROWS_PER_TILE = NPAD // NS
G = 128
EPW = 10240
NG = EPW // G
E_PAD = EPW * NW


def _spmm_body(with_cnt, d, y_hbm, src_hbm, dst_hbm, *rest):
    if with_cnt:
        (out_hbm, cnt_hbm, src_v, dst_v, rows_v, zcol_v, ones_v,
         acc_sh, cnt_sh, sem) = rest
    else:
        (out_hbm, src_v, dst_v, rows_v, acc_sh, sem) = rest
        cnt_hbm = zcol_v = ones_v = cnt_sh = None

    cid = lax.axis_index("c")
    sid = lax.axis_index("s")
    wid = sid * NC + cid

    zero16 = jnp.zeros((16,), jnp.float32)

    def zrow(i, _):
        def zcol(j, _):
            rows_v[i, pl.ds(j * 16, 16)] = zero16
            return 0
        return lax.fori_loop(0, d // 16, zcol, 0)

    lax.fori_loop(0, G, zrow, 0)

    if with_cnt:
        def zc(j, _):
            zcol_v[pl.ds(j * 16, 16)] = zero16
            return 0
        lax.fori_loop(0, ROWS_PER_TILE // 16, zc, 0)

        one16 = jnp.ones((16,), jnp.float32)

        def oc(j, _):
            ones_v[pl.ds(j * 16, 16)] = one16
            return 0
        lax.fori_loop(0, G // 16, oc, 0)

    for k in range(ROWS_PER_TILE // G):
        pltpu.sync_copy(rows_v, acc_sh.at[pl.ds(sid * ROWS_PER_TILE + k * G, G)])
    if with_cnt:
        pltpu.sync_copy(zcol_v, cnt_sh.at[pl.ds(sid * ROWS_PER_TILE, ROWS_PER_TILE)])
    plsc.subcore_barrier()

    def group(g, _):
        base = wid * EPW + g * G
        pltpu.sync_copy(src_hbm.at[pl.ds(base, G)], src_v)
        pltpu.sync_copy(dst_hbm.at[pl.ds(base, G)], dst_v)
        pltpu.async_copy(y_hbm.at[src_v], rows_v, sem).wait()
        pltpu.sync_copy(rows_v, acc_sh.at[dst_v], add=True)
        if with_cnt:
            pltpu.sync_copy(ones_v, cnt_sh.at[dst_v], add=True)
        return 0

    lax.fori_loop(0, NG, group, 0)
    plsc.subcore_barrier()

    for k in range(ROWS_PER_TILE // G):
        r0 = sid * ROWS_PER_TILE + k * G
        pltpu.sync_copy(acc_sh.at[pl.ds(r0, G)], out_hbm.at[cid, pl.ds(r0, G)])
    if with_cnt:
        pltpu.sync_copy(cnt_sh.at[pl.ds(sid * ROWS_PER_TILE, ROWS_PER_TILE)],
                        cnt_hbm.at[cid, pl.ds(sid * ROWS_PER_TILE, ROWS_PER_TILE)])


def _make_spmm(d, with_cnt):
    mesh = plsc.VectorSubcoreMesh(core_axis_name="c", subcore_axis_name="s")
    out_type = [jax.ShapeDtypeStruct((NC, NPAD, d), jnp.float32)]
    scratch = [
        pltpu.VMEM((G,), jnp.int32),
        pltpu.VMEM((G,), jnp.int32),
        pltpu.VMEM((G, d), jnp.float32),
        pltpu.VMEM_SHARED((NPAD, d), jnp.float32),
        pltpu.SemaphoreType.DMA,
    ]
    if with_cnt:
        out_type.append(jax.ShapeDtypeStruct((NC, NPAD), jnp.float32))
        scratch = [
            pltpu.VMEM((G,), jnp.int32),
            pltpu.VMEM((G,), jnp.int32),
            pltpu.VMEM((G, d), jnp.float32),
            pltpu.VMEM((ROWS_PER_TILE,), jnp.float32),
            pltpu.VMEM((G,), jnp.float32),
            pltpu.VMEM_SHARED((NPAD, d), jnp.float32),
            pltpu.VMEM_SHARED((NPAD,), jnp.float32),
            pltpu.SemaphoreType.DMA,
        ]
    return pl.kernel(
        functools.partial(_spmm_body, with_cnt, d),
        out_type=tuple(out_type),
        mesh=mesh,
        scratch_types=scratch,
        compiler_params=pltpu.CompilerParams(use_tc_tiling_on_sc=False),
    )


def _mm2_body(x_ref, wa_ref, wb_ref, ya_ref, yb_ref):
    x = x_ref[...]
    ya_ref[...] = jnp.dot(x, wa_ref[...], preferred_element_type=jnp.float32)
    yb_ref[...] = jnp.dot(x, wb_ref[...], preferred_element_type=jnp.float32)


def _layer1_body(ys_ref, part_ref, cntp_ref, b1_ref, wn2_ref, ws2_ref,
                 yn2_ref, ys2_ref, cnt_ref):
    s1 = part_ref[0, :N, :] + part_ref[1, :N, :]
    cnt = cntp_ref[0, :N, :] + cntp_ref[1, :N, :]
    mean = s1 / jnp.maximum(cnt, 1.0)
    h1 = jnp.maximum(ys_ref[...] + mean + b1_ref[...], 0.0)
    yn2_ref[...] = jnp.dot(h1, wn2_ref[...], preferred_element_type=jnp.float32)
    ys2_ref[...] = jnp.dot(h1, ws2_ref[...], preferred_element_type=jnp.float32)
    cnt_ref[...] = cnt


def _layer2_body(ys2_ref, part2_ref, cnt_ref, b2_ref, gw_ref, gb_ref, out_ref):
    s2 = part2_ref[0, :N, :] + part2_ref[1, :N, :]
    cnt = cnt_ref[...]
    h2 = ys2_ref[...] + s2 / jnp.maximum(cnt, 1.0) + b2_ref[...]
    g = jnp.sum(h2 * gw_ref[...], axis=1, keepdims=True) + gb_ref[0, 0]
    m = jnp.max(g)
    e = jnp.exp(g - m)
    z = jnp.sum(e)
    out_ref[...] = jnp.sum(h2 * (e / z), axis=0, keepdims=True)


def kernel(features, edge_index, W_self1, W_neigh1, b1, W_self2, W_neigh2,
           b2, gate_W, gate_b):
    src = edge_index[0].astype(jnp.int32)
    dst = edge_index[1].astype(jnp.int32)
    pad = E_PAD - E
    src_p = jnp.concatenate([src, jnp.zeros((pad,), jnp.int32)])
    dst_p = jnp.concatenate([dst, jnp.full((pad,), N, jnp.int32)])

    f32 = jnp.float32
    sds = jax.ShapeDtypeStruct

    yn1, ys1 = pl.pallas_call(
        _mm2_body,
        out_shape=(sds((N, D_HID), f32), sds((N, D_HID), f32)),
    )(features, W_neigh1, W_self1)

    part1, cntp = _make_spmm(D_HID, True)(yn1, src_p, dst_p)
    cntp = cntp.reshape(NC, NPAD, 1)

    yn2, ys2, cnt = pl.pallas_call(
        _layer1_body,
        out_shape=(sds((N, D_OUT), f32), sds((N, D_OUT), f32), sds((N, 1), f32)),
    )(ys1, part1, cntp, b1.reshape(1, D_HID), W_neigh2, W_self2)

    (part2,) = _make_spmm(D_OUT, False)(yn2, src_p, dst_p)

    out = pl.pallas_call(
        _layer2_body,
        out_shape=sds((1, D_OUT), f32),
    )(ys2, part2, cnt, b2.reshape(1, D_OUT), gate_W.reshape(1, D_OUT),
      gate_b.reshape(1, 1))
    return out
